# baseline (speedup 1.0000x reference)
import functools

import jax
import jax.numpy as jnp
from jax import lax
from jax.experimental import pallas as pl
from jax.experimental.pallas import tpu as pltpu
from jax.experimental.pallas import tpu_sc as plsc

NUM_WORKERS = 32
IDX_CHUNK = 128
LANES = 16


def _gather_pair_packed(uid, iid, user_table, item_table):
    batch = uid.shape[0]
    chunk = IDX_CHUNK
    embed = user_table.shape[1]
    rows_per_w = batch // NUM_WORKERS
    nch = rows_per_w // chunk
    pk_sub = chunk // 2
    ngrp = embed // LANES

    mesh = plsc.VectorSubcoreMesh(core_axis_name="c", subcore_axis_name="s")

    @functools.partial(
        pl.kernel,
        mesh=mesh,
        compiler_params=pltpu.CompilerParams(needs_layout_passes=False),
        out_type=(
            jax.ShapeDtypeStruct((batch // 2, embed), jnp.int32),
            jax.ShapeDtypeStruct((batch // 2, embed), jnp.int32),
        ),
        scratch_types=[
            pltpu.VMEM((nch, chunk), jnp.int32),
            pltpu.VMEM((nch, chunk), jnp.int32),
            pltpu.VMEM((rows_per_w, embed), jnp.float32),
            pltpu.VMEM((2 * pk_sub, embed), jnp.int32),
            pltpu.SemaphoreType.DMA,
            pltpu.SemaphoreType.DMA,
            pltpu.SemaphoreType.DMA,
            pltpu.SemaphoreType.DMA,
        ],
    )
    def gather_kernel(uid_hbm, iid_hbm, ut_hbm, it_hbm, out_u, out_i,
                      uidx_v, iidx_v, stg, pkb, sem_u, sem_i, sem_w0, sem_w1):
        wid = lax.axis_index("s") * 2 + lax.axis_index("c")
        id_base = wid * rows_per_w
        out_base = wid * (rows_per_w // 2)
        id_cps = [
            pltpu.async_copy(hbm.at[pl.ds(id_base + j * chunk, chunk)],
                             v.at[j], sem_w0)
            for hbm, v in ((uid_hbm, uidx_v), (iid_hbm, iidx_v))
            for j in range(nch)
        ]
        for c in id_cps:
            c.wait()
        cps_u = [
            pltpu.async_copy(ut_hbm.at[uidx_v.at[j]],
                             stg.at[pl.ds(j * chunk, chunk)], sem_u)
            for j in range(nch)
        ]
        cps_i = [None] * nch
        writes = []
        sem_ws = (sem_w0, sem_w1)

        def pack_sub(src_row, dst_row):
            @plsc.parallel_loop(0, pk_sub, 1, unroll=4)
            def _(q):
                for g in range(ngrp):
                    a = stg[src_row + 2 * q, pl.ds(g * LANES, LANES)]
                    b = stg[src_row + 2 * q + 1, pl.ds(g * LANES, LANES)]
                    p = plsc.pack(a, b, format=plsc.PackFormat.INTERLEAVED)
                    pkb[dst_row + q, pl.ds(g * LANES, LANES)] = plsc.bitcast(
                        p, jnp.int32)

        for t in range(2 * nch):
            j = t % nch
            half = t % 2
            if t < nch:
                cps_u[j].wait()
            else:
                cps_i[j].wait()
            if t >= 2:
                writes[t - 2].wait()
            pack_sub(j * chunk, half * pk_sub)
            if t < nch:
                cps_i[j] = pltpu.async_copy(
                    it_hbm.at[iidx_v.at[j]],
                    stg.at[pl.ds(j * chunk, chunk)], sem_i)
            out_ref = out_u if t < nch else out_i
            writes.append(pltpu.async_copy(
                pkb.at[pl.ds(half * pk_sub, pk_sub)],
                out_ref.at[pl.ds(out_base + j * pk_sub, pk_sub)],
                sem_ws[half]))
        writes[-2].wait()
        writes[-1].wait()

    return gather_kernel(uid, iid, user_table, item_table)


def _mlp_body(upk_ref, ipk_ref, w1_ref, b1_ref, w2_ref, b2_ref,
              w3_ref, b3_ref, o_ref):
    ue = pltpu.bitcast(upk_ref[...], jnp.bfloat16)
    ie = pltpu.bitcast(ipk_ref[...], jnp.bfloat16)
    embed = ue.shape[1]
    h = jnp.dot(ue, w1_ref[0:embed, :], preferred_element_type=jnp.float32)
    h = h + jnp.dot(ie, w1_ref[embed:2 * embed, :],
                    preferred_element_type=jnp.float32)
    h1 = jnp.maximum(h + b1_ref[...], 0.0)
    h2 = jnp.maximum(
        jnp.dot(h1, w2_ref[...], preferred_element_type=jnp.float32)
        + b2_ref[...], 0.0)
    ot = lax.dot_general(w3_ref[...], h2, (((0,), (1,)), ((), ())),
                         preferred_element_type=jnp.float32)
    o_ref[...] = jnp.maximum(ot + b3_ref[...], 0.0)[None]


def _mlp(upk, ipk, w1, b1, w2, b2, w3, b3, block=1024):
    pk_rows, embed = upk.shape
    batch = pk_rows * 2
    grid = batch // block
    full = lambda shape: pl.BlockSpec(shape, lambda i: (0, 0))
    return pl.pallas_call(
        _mlp_body,
        grid=(grid,),
        in_specs=[
            pl.BlockSpec((block // 2, embed), lambda i: (i, 0)),
            pl.BlockSpec((block // 2, embed), lambda i: (i, 0)),
            full(w1.shape),
            full(b1.shape),
            full(w2.shape),
            full(b2.shape),
            full(w3.shape),
            full(b3.shape),
        ],
        out_specs=pl.BlockSpec((1, 1, block), lambda i: (i, 0, 0)),
        out_shape=jax.ShapeDtypeStruct((grid, 1, block), jnp.float32),
    )(upk, ipk, w1, b1, w2, b2, w3, b3)


def kernel(user_id, item_id, user_table, item_table, W1, b1, W2, b2, W3, b3):
    w1_bf = W1.astype(jnp.bfloat16)
    b1r = b1.reshape(1, -1)
    b2r = b2.reshape(1, -1)
    b3r = b3.reshape(1, 1)
    upk, ipk = _gather_pair_packed(user_id.astype(jnp.int32),
                                   item_id.astype(jnp.int32),
                                   user_table, item_table)
    return _mlp(upk, ipk, w1_bf, b1r, W2, b2r, W3, b3r).reshape(-1)


# speedup vs baseline: 1.0469x; 1.0469x over previous
import functools

import jax
import jax.numpy as jnp
from jax import lax
from jax.experimental import pallas as pl
from jax.experimental.pallas import tpu as pltpu
from jax.experimental.pallas import tpu_sc as plsc

NUM_WORKERS = 32
IDX_CHUNK = 128
LANES = 16


def _gather_pair_packed(uid, iid, user_table, item_table):
    batch = uid.shape[0]
    chunk = IDX_CHUNK
    embed = user_table.shape[1]
    rows_per_w = batch // NUM_WORKERS
    nch = rows_per_w // chunk
    pk_sub = chunk // 2
    ngrp = embed // LANES

    mesh = plsc.VectorSubcoreMesh(core_axis_name="c", subcore_axis_name="s")

    @functools.partial(
        pl.kernel,
        mesh=mesh,
        compiler_params=pltpu.CompilerParams(needs_layout_passes=False),
        out_type=(
            jax.ShapeDtypeStruct((batch // 2, embed), jnp.int32),
            jax.ShapeDtypeStruct((batch // 2, embed), jnp.int32),
        ),
        scratch_types=[
            pltpu.VMEM((nch, chunk), jnp.int32),
            pltpu.VMEM((nch, chunk), jnp.int32),
            pltpu.VMEM((rows_per_w, embed), jnp.float32),
            pltpu.VMEM((2 * pk_sub, embed), jnp.int32),
            pltpu.SemaphoreType.DMA,
            pltpu.SemaphoreType.DMA,
            pltpu.SemaphoreType.DMA,
            pltpu.SemaphoreType.DMA,
        ],
    )
    def gather_kernel(uid_hbm, iid_hbm, ut_hbm, it_hbm, out_u, out_i,
                      uidx_v, iidx_v, stg, pkb, sem_u, sem_i, sem_w0, sem_w1):
        wid = lax.axis_index("s") * 2 + lax.axis_index("c")
        id_base = wid * rows_per_w
        out_base = wid * (rows_per_w // 2)
        id_cps = [
            pltpu.async_copy(hbm.at[pl.ds(id_base + j * chunk, chunk)],
                             v.at[j], sem_w0)
            for hbm, v in ((uid_hbm, uidx_v), (iid_hbm, iidx_v))
            for j in range(nch)
        ]
        for c in id_cps:
            c.wait()
        cps_u = [
            pltpu.async_copy(ut_hbm.at[uidx_v.at[j]],
                             stg.at[pl.ds(j * chunk, chunk)], sem_u)
            for j in range(nch)
        ]
        cps_i = [None] * nch
        writes = []
        sem_ws = (sem_w0, sem_w1)

        def pack_sub(src_row, dst_row):
            @plsc.parallel_loop(0, pk_sub, 1, unroll=8)
            def _(q):
                for g in range(ngrp):
                    a = stg[src_row + 2 * q, pl.ds(g * LANES, LANES)]
                    b = stg[src_row + 2 * q + 1, pl.ds(g * LANES, LANES)]
                    p = plsc.pack(a, b, format=plsc.PackFormat.INTERLEAVED)
                    pkb[dst_row + q, pl.ds(g * LANES, LANES)] = plsc.bitcast(
                        p, jnp.int32)

        for t in range(2 * nch):
            j = t % nch
            half = t % 2
            if t < nch:
                cps_u[j].wait()
            else:
                cps_i[j].wait()
            if t >= 2:
                writes[t - 2].wait()
            pack_sub(j * chunk, half * pk_sub)
            if t < nch:
                cps_i[j] = pltpu.async_copy(
                    it_hbm.at[iidx_v.at[j]],
                    stg.at[pl.ds(j * chunk, chunk)], sem_i)
            out_ref = out_u if t < nch else out_i
            writes.append(pltpu.async_copy(
                pkb.at[pl.ds(half * pk_sub, pk_sub)],
                out_ref.at[pl.ds(out_base + j * pk_sub, pk_sub)],
                sem_ws[half]))
        writes[-2].wait()
        writes[-1].wait()

    return gather_kernel(uid, iid, user_table, item_table)


def _mlp_body(upk_ref, ipk_ref, w1_ref, b1_ref, w2_ref, b2_ref,
              w3_ref, b3_ref, o_ref):
    ue = pltpu.bitcast(upk_ref[...], jnp.bfloat16)
    ie = pltpu.bitcast(ipk_ref[...], jnp.bfloat16)
    embed = ue.shape[1]
    h = jnp.dot(ue, w1_ref[0:embed, :], preferred_element_type=jnp.float32)
    h = h + jnp.dot(ie, w1_ref[embed:2 * embed, :],
                    preferred_element_type=jnp.float32)
    h1 = jnp.maximum(h + b1_ref[...], 0.0)
    h2 = jnp.maximum(
        jnp.dot(h1, w2_ref[...], preferred_element_type=jnp.float32)
        + b2_ref[...], 0.0)
    ot = lax.dot_general(w3_ref[...], h2, (((0,), (1,)), ((), ())),
                         preferred_element_type=jnp.float32)
    o_ref[...] = jnp.maximum(ot + b3_ref[...], 0.0)[None]


def _mlp(upk, ipk, w1, b1, w2, b2, w3, b3, block=2048):
    pk_rows, embed = upk.shape
    batch = pk_rows * 2
    grid = batch // block
    full = lambda shape: pl.BlockSpec(shape, lambda i: (0, 0))
    return pl.pallas_call(
        _mlp_body,
        grid=(grid,),
        in_specs=[
            pl.BlockSpec((block // 2, embed), lambda i: (i, 0)),
            pl.BlockSpec((block // 2, embed), lambda i: (i, 0)),
            full(w1.shape),
            full(b1.shape),
            full(w2.shape),
            full(b2.shape),
            full(w3.shape),
            full(b3.shape),
        ],
        out_specs=pl.BlockSpec((1, 1, block), lambda i: (i, 0, 0)),
        out_shape=jax.ShapeDtypeStruct((grid, 1, block), jnp.float32),
    )(upk, ipk, w1, b1, w2, b2, w3, b3)


def kernel(user_id, item_id, user_table, item_table, W1, b1, W2, b2, W3, b3):
    w1_bf = W1.astype(jnp.bfloat16)
    b1r = b1.reshape(1, -1)
    b2r = b2.reshape(1, -1)
    b3r = b3.reshape(1, 1)
    upk, ipk = _gather_pair_packed(user_id.astype(jnp.int32),
                                   item_id.astype(jnp.int32),
                                   user_table, item_table)
    return _mlp(upk, ipk, w1_bf, b1r, W2, b2r, W3, b3r).reshape(-1)


# speedup vs baseline: 1.1068x; 1.0572x over previous
import functools

import jax
import jax.numpy as jnp
from jax import lax
from jax.experimental import pallas as pl
from jax.experimental.pallas import tpu as pltpu
from jax.experimental.pallas import tpu_sc as plsc

NUM_WORKERS = 32
IDX_CHUNK = 128
LANES = 16


def _gather_pair_packed(uid, iid, user_table, item_table):
    batch = uid.shape[0]
    chunk = IDX_CHUNK
    embed = user_table.shape[1]
    rows_per_w = batch // NUM_WORKERS
    nch = rows_per_w // chunk
    pk_sub = chunk // 2
    ngrp = embed // LANES

    mesh = plsc.VectorSubcoreMesh(core_axis_name="c", subcore_axis_name="s")

    @functools.partial(
        pl.kernel,
        mesh=mesh,
        compiler_params=pltpu.CompilerParams(needs_layout_passes=False),
        out_type=(
            jax.ShapeDtypeStruct((batch // 2, embed), jnp.int32),
            jax.ShapeDtypeStruct((batch // 2, embed), jnp.int32),
        ),
        scratch_types=[
            pltpu.VMEM((nch, chunk), jnp.int32),
            pltpu.VMEM((nch, chunk), jnp.int32),
            pltpu.VMEM((rows_per_w, embed), jnp.float32),
            pltpu.VMEM((2 * pk_sub, embed), jnp.int32),
            pltpu.SemaphoreType.DMA,
            pltpu.SemaphoreType.DMA,
            pltpu.SemaphoreType.DMA,
            pltpu.SemaphoreType.DMA,
        ],
    )
    def gather_kernel(uid_hbm, iid_hbm, ut_hbm, it_hbm, out_u, out_i,
                      uidx_v, iidx_v, stg, pkb, sem_u, sem_i, sem_w0, sem_w1):
        wid = lax.axis_index("s") * 2 + lax.axis_index("c")
        id_base = wid * rows_per_w
        out_base = wid * (rows_per_w // 2)
        id_cps = [
            pltpu.async_copy(hbm.at[pl.ds(id_base + j * chunk, chunk)],
                             v.at[j], sem_w0)
            for hbm, v in ((uid_hbm, uidx_v), (iid_hbm, iidx_v))
            for j in range(nch)
        ]
        for c in id_cps:
            c.wait()
        cps_u = [
            pltpu.async_copy(ut_hbm.at[uidx_v.at[j]],
                             stg.at[pl.ds(j * chunk, chunk)], sem_u)
            for j in range(nch)
        ]
        cps_i = [None] * nch
        writes = []
        sem_ws = (sem_w0, sem_w1)

        def pack_sub(src_row, dst_row):
            @plsc.parallel_loop(0, pk_sub, 1, unroll=4)
            def _(q):
                for g in range(ngrp):
                    a = stg[src_row + 2 * q, pl.ds(g * LANES, LANES)]
                    b = stg[src_row + 2 * q + 1, pl.ds(g * LANES, LANES)]
                    p = plsc.pack(a, b, format=plsc.PackFormat.INTERLEAVED)
                    pkb[dst_row + q, pl.ds(g * LANES, LANES)] = plsc.bitcast(
                        p, jnp.int32)

        for t in range(2 * nch):
            j = t % nch
            half = t % 2
            if t < nch:
                cps_u[j].wait()
            else:
                cps_i[j].wait()
            if t >= 2:
                writes[t - 2].wait()
            pack_sub(j * chunk, half * pk_sub)
            if t < nch:
                cps_i[j] = pltpu.async_copy(
                    it_hbm.at[iidx_v.at[j]],
                    stg.at[pl.ds(j * chunk, chunk)], sem_i)
            out_ref = out_u if t < nch else out_i
            writes.append(pltpu.async_copy(
                pkb.at[pl.ds(half * pk_sub, pk_sub)],
                out_ref.at[pl.ds(out_base + j * pk_sub, pk_sub)],
                sem_ws[half]))
        writes[-2].wait()
        writes[-1].wait()

    return gather_kernel(uid, iid, user_table, item_table)


def _mlp_body(upk_ref, ipk_ref, w1_ref, b1_ref, w2_ref, b2_ref,
              w3_ref, b3_ref, o_ref):
    ue = pltpu.bitcast(upk_ref[...], jnp.bfloat16)
    ie = pltpu.bitcast(ipk_ref[...], jnp.bfloat16)
    embed = ue.shape[1]
    h = jnp.dot(ue, w1_ref[0:embed, :], preferred_element_type=jnp.float32)
    h = h + jnp.dot(ie, w1_ref[embed:2 * embed, :],
                    preferred_element_type=jnp.float32)
    h1 = jnp.maximum(h + b1_ref[...], 0.0)
    h2 = jnp.maximum(
        jnp.dot(h1, w2_ref[...], preferred_element_type=jnp.float32)
        + b2_ref[...], 0.0)
    ot = lax.dot_general(w3_ref[...], h2, (((0,), (1,)), ((), ())),
                         preferred_element_type=jnp.float32)
    o_ref[...] = jnp.maximum(ot + b3_ref[...], 0.0)[None]


def _mlp(upk, ipk, w1, b1, w2, b2, w3, b3, block=2048):
    pk_rows, embed = upk.shape
    batch = pk_rows * 2
    grid = batch // block
    full = lambda shape: pl.BlockSpec(shape, lambda i: (0, 0))
    return pl.pallas_call(
        _mlp_body,
        grid=(grid,),
        in_specs=[
            pl.BlockSpec((block // 2, embed), lambda i: (i, 0)),
            pl.BlockSpec((block // 2, embed), lambda i: (i, 0)),
            full(w1.shape),
            full(b1.shape),
            full(w2.shape),
            full(b2.shape),
            full(w3.shape),
            full(b3.shape),
        ],
        out_specs=pl.BlockSpec((1, 1, block), lambda i: (i, 0, 0)),
        out_shape=jax.ShapeDtypeStruct((grid, 1, block), jnp.float32),
    )(upk, ipk, w1, b1, w2, b2, w3, b3)


def kernel(user_id, item_id, user_table, item_table, W1, b1, W2, b2, W3, b3):
    w1_bf = W1.astype(jnp.bfloat16)
    b1r = b1.reshape(1, -1)
    b2r = b2.reshape(1, -1)
    b3r = b3.reshape(1, 1)
    upk, ipk = _gather_pair_packed(user_id.astype(jnp.int32),
                                   item_id.astype(jnp.int32),
                                   user_table, item_table)
    return _mlp(upk, ipk, w1_bf, b1r, W2, b2r, W3, b3r).reshape(-1)
